# Initial kernel scaffold; baseline (speedup 1.0000x reference)
import functools
import jax
import jax.numpy as jnp
from jax.experimental import pallas as pl
from jax.experimental.pallas import tpu as pltpu

N0, N1, N2, B = 10000, 2500, 625, 16
E0, E1 = 160000, 40000
D_IN, D_MID, D_H = 768, 512, 256


def _encoder_body(x_ref, w1_ref, b1_ref, w2_ref, b2_ref, alpha_ref, w0_ref,
                  dinv_ref, out_ref):
    h = jnp.dot(x_ref[...], w1_ref[...], preferred_element_type=jnp.float32)
    h = h + b1_ref[...]
    h = jnp.where(h >= 0, h, 0.01 * h)
    h = jnp.dot(h, w2_ref[...], preferred_element_type=jnp.float32)
    h = h + b2_ref[...]
    h = jnp.where(h >= 0, h, alpha_ref[...] * h)
    p = jnp.dot(h, w0_ref[...], preferred_element_type=jnp.float32)
    out_ref[...] = p * dinv_ref[...]


def _encoder(x, w1, b1, w2, b2, alpha, w0, dinv):
    blk = 1000
    grid = (N0 // blk,)
    return pl.pallas_call(
        _encoder_body,
        grid=grid,
        in_specs=[
            pl.BlockSpec((blk, D_IN), lambda i: (i, 0)),
            pl.BlockSpec((D_IN, D_MID), lambda i: (0, 0)),
            pl.BlockSpec((1, D_MID), lambda i: (0, 0)),
            pl.BlockSpec((D_MID, D_H), lambda i: (0, 0)),
            pl.BlockSpec((1, D_H), lambda i: (0, 0)),
            pl.BlockSpec((1, D_H), lambda i: (0, 0)),
            pl.BlockSpec((D_H, D_H), lambda i: (0, 0)),
            pl.BlockSpec((blk, 1), lambda i: (i, 0)),
        ],
        out_specs=pl.BlockSpec((blk, D_H), lambda i: (i, 0)),
        out_shape=jax.ShapeDtypeStruct((N0, D_H), jnp.float32),
    )(x, w1, b1.reshape(1, -1), w2, b2.reshape(1, -1), alpha.reshape(1, -1),
      w0, dinv.reshape(-1, 1))


def kernel(x, edge_index_l0, sep1_assign, edge_index_l1, sep2_assign,
           batch_l1, batch_l2,
           enc_W1, enc_b1, enc_W2, enc_b2, enc_alpha,
           gcn_W0, gcn_b0, gcn_W1, gcn_b1,
           cls_W1, cls_b1, cls_W2, cls_b2):
    src0, dst0 = edge_index_l0[0], edge_index_l0[1]
    src1, dst1 = edge_index_l1[0], edge_index_l1[1]

    deg0 = jnp.zeros((N0,), jnp.float32).at[dst0].add(1.0) + 1.0
    deg1 = jnp.zeros((N1,), jnp.float32).at[dst1].add(1.0) + 1.0
    dinv0 = deg0 ** -0.5
    dinv1 = deg1 ** -0.5

    u0 = _encoder(x, enc_W1, enc_b1, enc_W2, enc_b2, enc_alpha, gcn_W0, dinv0)

    acc0 = jnp.zeros((N0, D_H), jnp.float32).at[dst0].add(u0[src0])
    g0 = jax.nn.relu(dinv0[:, None] * (acc0 + u0) + gcn_b0)

    s1 = jnp.zeros((N1, D_H), jnp.float32).at[sep1_assign].add(g0)
    xs0 = jax.nn.relu(s1)

    u1 = dinv1[:, None] * (xs0 @ gcn_W1)
    acc1 = jnp.zeros((N1, D_H), jnp.float32).at[dst1].add(u1[src1])
    g1 = jax.nn.relu(dinv1[:, None] * (acc1 + u1) + gcn_b1)

    s2 = jnp.zeros((N2, D_H), jnp.float32).at[sep2_assign].add(g1)
    xs1 = jax.nn.relu(s2)

    p0 = jax.ops.segment_sum(xs0, batch_l1, num_segments=B)
    p1 = jax.ops.segment_sum(xs1, batch_l2, num_segments=B)
    z = jnp.concatenate([p0, p1], axis=1)
    z = jax.nn.relu(z @ cls_W1 + cls_b1)
    return z @ cls_W2 + cls_b2


# speedup vs baseline: 2.7209x; 2.7209x over previous
import functools
import jax
import jax.numpy as jnp
from jax.experimental import pallas as pl
from jax.experimental.pallas import tpu as pltpu

N0, N1, N2, B = 10000, 2500, 625, 16
E0, E1 = 160000, 40000
D_IN, D_MID, D_H = 768, 512, 256


_HI = jax.lax.Precision.HIGHEST


def _encoder_body(x_ref, w1_ref, b1_ref, w2_ref, b2_ref, alpha_ref, w0_ref,
                  dinv_ref, out_ref):
    h = jnp.dot(x_ref[...], w1_ref[...], preferred_element_type=jnp.float32,
                precision=_HI)
    h = h + b1_ref[...]
    h = jnp.where(h >= 0, h, 0.01 * h)
    h = jnp.dot(h, w2_ref[...], preferred_element_type=jnp.float32,
                precision=_HI)
    h = h + b2_ref[...]
    h = jnp.where(h >= 0, h, alpha_ref[...] * h)
    p = jnp.dot(h, w0_ref[...], preferred_element_type=jnp.float32,
                precision=_HI)
    out_ref[...] = p * dinv_ref[...]


def _encoder(x, w1, b1, w2, b2, alpha, w0, dinv):
    blk = 1000
    grid = (N0 // blk,)
    return pl.pallas_call(
        _encoder_body,
        grid=grid,
        in_specs=[
            pl.BlockSpec((blk, D_IN), lambda i: (i, 0)),
            pl.BlockSpec((D_IN, D_MID), lambda i: (0, 0)),
            pl.BlockSpec((1, D_MID), lambda i: (0, 0)),
            pl.BlockSpec((D_MID, D_H), lambda i: (0, 0)),
            pl.BlockSpec((1, D_H), lambda i: (0, 0)),
            pl.BlockSpec((1, D_H), lambda i: (0, 0)),
            pl.BlockSpec((D_H, D_H), lambda i: (0, 0)),
            pl.BlockSpec((blk, 1), lambda i: (i, 0)),
        ],
        out_specs=pl.BlockSpec((blk, D_H), lambda i: (i, 0)),
        out_shape=jax.ShapeDtypeStruct((N0, D_H), jnp.float32),
    )(x, w1, b1.reshape(1, -1), w2, b2.reshape(1, -1), alpha.reshape(1, -1),
      w0, dinv.reshape(-1, 1))


def kernel(x, edge_index_l0, sep1_assign, edge_index_l1, sep2_assign,
           batch_l1, batch_l2,
           enc_W1, enc_b1, enc_W2, enc_b2, enc_alpha,
           gcn_W0, gcn_b0, gcn_W1, gcn_b1,
           cls_W1, cls_b1, cls_W2, cls_b2):
    src0, dst0 = edge_index_l0[0], edge_index_l0[1]
    src1, dst1 = edge_index_l1[0], edge_index_l1[1]

    deg0 = jnp.zeros((N0,), jnp.float32).at[dst0].add(1.0) + 1.0
    deg1 = jnp.zeros((N1,), jnp.float32).at[dst1].add(1.0) + 1.0
    dinv0 = deg0 ** -0.5
    dinv1 = deg1 ** -0.5

    u0 = _encoder(x, enc_W1, enc_b1, enc_W2, enc_b2, enc_alpha, gcn_W0, dinv0)

    acc0 = jnp.zeros((N0, D_H), jnp.float32).at[dst0].add(u0[src0])
    g0 = jax.nn.relu(dinv0[:, None] * (acc0 + u0) + gcn_b0)

    s1 = jnp.zeros((N1, D_H), jnp.float32).at[sep1_assign].add(g0)
    xs0 = jax.nn.relu(s1)

    u1 = dinv1[:, None] * (xs0 @ gcn_W1)
    acc1 = jnp.zeros((N1, D_H), jnp.float32).at[dst1].add(u1[src1])
    g1 = jax.nn.relu(dinv1[:, None] * (acc1 + u1) + gcn_b1)

    s2 = jnp.zeros((N2, D_H), jnp.float32).at[sep2_assign].add(g1)
    xs1 = jax.nn.relu(s2)

    p0 = jax.ops.segment_sum(xs0, batch_l1, num_segments=B)
    p1 = jax.ops.segment_sum(xs1, batch_l2, num_segments=B)
    z = jnp.concatenate([p0, p1], axis=1)
    z = jax.nn.relu(z @ cls_W1 + cls_b1)
    return z @ cls_W2 + cls_b2


# speedup vs baseline: 4.1671x; 1.5315x over previous
import functools
import jax
import jax.numpy as jnp
from jax import lax
from jax.experimental import pallas as pl
from jax.experimental.pallas import tpu as pltpu
from jax.experimental.pallas import tpu_sc as plsc

N0, N1, N2, B = 10000, 2500, 625, 16
E0, E1 = 160000, 40000
D_IN, D_MID, D_H = 768, 512, 256

NC, NS, L = 2, 16, 16

_HI = jax.lax.Precision.HIGHEST


def _round16(n):
    return (n + 15) // 16 * 16


def _chunks_of(total, step):
    out = []
    while total > 0:
        out.append(min(step, total))
        total -= out[-1]
    return out


def _pad_idx(idx, pad_to, pad_base):
    n = idx.shape[0]
    pad = pad_base + (jnp.arange(pad_to - n, dtype=jnp.int32) % 8)
    return jnp.concatenate([idx.astype(jnp.int32), pad])


def _sc_mesh():
    return plsc.VectorSubcoreMesh(core_axis_name="c", subcore_axis_name="s",
                                  num_cores=NC, num_subcores=NS)


K_FIRE = 64


def _make_scatter_kernel(n_dstp, e_pad, with_gather=True):
    K = K_FIRE
    D = D_H if with_gather else L
    TR = n_dstp // (NC * NS)
    assert TR % 8 == 0 and e_pad % K == 0

    scratch = [
        pltpu.VMEM((1, K), jnp.int32),
        pltpu.VMEM((48,), jnp.int32),
        pltpu.VMEM((TR + 8, D), jnp.float32),
        pltpu.SemaphoreType.DMA,
    ]
    if with_gather:
        scratch = [pltpu.VMEM((1, K), jnp.int32),
                   pltpu.VMEM((K, D), jnp.float32),
                   ] + scratch

    def body(*refs):
        if with_gather:
            (g_hbm, s_hbm, ptr_hbm, v_hbm, out_hbm,
             g_stage, rows, s_chunk, ptr_buf, acc, sem) = refs
        else:
            (s_hbm, ptr_hbm, out_hbm,
             s_chunk, ptr_buf, acc, sem) = refs
        c = lax.axis_index("c")
        s = lax.axis_index("s")
        w = c * NS + s
        base = w * TR

        def zrow(i, _):
            for dd in range(D // L):
                acc[i, pl.ds(dd * L, L)] = jnp.zeros((L,), jnp.float32)
            return 0
        lax.fori_loop(0, TR, zrow, 0)

        pltpu.sync_copy(ptr_hbm, ptr_buf)
        p0 = ptr_buf[pl.ds(w, L)][0]
        p1 = ptr_buf[pl.ds(w + 1, L)][0]
        j0 = p0 >> 6
        j1 = (p1 + K - 1) >> 6
        ones_vec = jnp.ones((L,), jnp.float32)

        def chunk(j, _):
            pltpu.sync_copy(s_hbm.at[j], s_chunk)
            if with_gather:
                pltpu.sync_copy(g_hbm.at[j], g_stage)
                pltpu.async_copy(v_hbm.at[g_stage.at[0]], rows, sem).wait()

            for t in range(K // L):
                dvec = s_chunk[0, pl.ds(t * L, L)]
                for lane in range(L):
                    k = t * L + lane
                    e = j * K + k
                    valid = (e >= p0) & (e < p1)
                    rr = jnp.where(valid, dvec[lane] - base,
                                   TR + (lane & 7))
                    if with_gather:
                        for dt in range(D // L):
                            plsc.addupdate(acc.at[rr, pl.ds(dt * L, L)],
                                           rows[k, pl.ds(dt * L, L)])
                    else:
                        plsc.addupdate(acc.at[rr, pl.ds(0, L)], ones_vec)
            return 0
        lax.fori_loop(j0, j1, chunk, 0)

        pltpu.sync_copy(acc.at[pl.ds(0, TR)],
                        out_hbm.at[pl.ds(base, TR)])

    return functools.partial(
        pl.kernel,
        out_type=jax.ShapeDtypeStruct((n_dstp, D), jnp.float32),
        mesh=_sc_mesh(),
        scratch_types=scratch,
    )(body)


def _encoder_body(x_ref, w1_ref, b1_ref, w2_ref, b2_ref, alpha_ref, w0_ref,
                  dinv_ref, out_ref):
    h = jnp.dot(x_ref[...], w1_ref[...], preferred_element_type=jnp.float32,
                precision=_HI)
    h = h + b1_ref[...]
    h = jnp.where(h >= 0, h, 0.01 * h)
    h = jnp.dot(h, w2_ref[...], preferred_element_type=jnp.float32,
                precision=_HI)
    h = h + b2_ref[...]
    h = jnp.where(h >= 0, h, alpha_ref[...] * h)
    p = jnp.dot(h, w0_ref[...], preferred_element_type=jnp.float32,
                precision=_HI)
    out_ref[...] = p * dinv_ref[...]


def _encoder(x, w1, b1, w2, b2, alpha, w0, dinv):
    blk = 1000
    return pl.pallas_call(
        _encoder_body,
        grid=(N0 // blk,),
        in_specs=[
            pl.BlockSpec((blk, D_IN), lambda i: (i, 0)),
            pl.BlockSpec((D_IN, D_MID), lambda i: (0, 0)),
            pl.BlockSpec((1, D_MID), lambda i: (0, 0)),
            pl.BlockSpec((D_MID, D_H), lambda i: (0, 0)),
            pl.BlockSpec((1, D_H), lambda i: (0, 0)),
            pl.BlockSpec((1, D_H), lambda i: (0, 0)),
            pl.BlockSpec((D_H, D_H), lambda i: (0, 0)),
            pl.BlockSpec((blk, 1), lambda i: (i, 0)),
        ],
        out_specs=pl.BlockSpec((blk, D_H), lambda i: (i, 0)),
        out_shape=jax.ShapeDtypeStruct((N0, D_H), jnp.float32),
    )(x, w1, b1.reshape(1, -1), w2, b2.reshape(1, -1), alpha.reshape(1, -1),
      w0, dinv.reshape(-1, 1))


def _gcn0_post_body(acc_ref, u_ref, dinv_ref, b_ref, assign_ref, s1_ref):
    i = pl.program_id(0)
    blk = acc_ref.shape[0]
    n1p = s1_ref.shape[0]
    dv = dinv_ref[pl.ds(i * blk, blk), :]
    g0 = jnp.maximum(dv * (acc_ref[...] + u_ref[...]) + b_ref[...], 0.0)
    a1 = assign_ref[0, 0, :]
    m = (a1[:, None] == lax.broadcasted_iota(jnp.int32, (blk, n1p), 1))
    contrib = lax.dot_general(m.astype(jnp.float32), g0,
                              (((0,), (0,)), ((), ())),
                              preferred_element_type=jnp.float32,
                              precision=_HI)

    @pl.when(i == 0)
    def _():
        s1_ref[...] = jnp.zeros_like(s1_ref)
    s1_ref[...] += contrib


def _gcn0_post_sep1(acc_flat, u0, dinv0, b0, sep1_assign, n1p):
    blk = 1000
    return pl.pallas_call(
        _gcn0_post_body,
        grid=(N0 // blk,),
        in_specs=[
            pl.BlockSpec((blk, D_H), lambda i: (i, 0)),
            pl.BlockSpec((blk, D_H), lambda i: (i, 0)),
            pl.BlockSpec((N0, 1), lambda i: (0, 0)),
            pl.BlockSpec((1, D_H), lambda i: (0, 0)),
            pl.BlockSpec((1, 1, blk), lambda i: (i, 0, 0)),
        ],
        out_specs=pl.BlockSpec((n1p, D_H), lambda i: (0, 0)),
        out_shape=jax.ShapeDtypeStruct((n1p, D_H), jnp.float32),
    )(acc_flat, u0, dinv0.reshape(-1, 1), b0.reshape(1, -1),
      sep1_assign.astype(jnp.int32).reshape(N0 // blk, 1, blk))


def _gcn1_post_body(a_ref, u_ref, dinv_ref, b_ref, assign_ref, s2_ref):
    i = pl.program_id(0)
    blk = a_ref.shape[0]
    dv = dinv_ref[pl.ds(i * blk, blk), :]
    g1 = jnp.maximum(dv * (a_ref[...] + u_ref[...]) + b_ref[...], 0.0)
    a2 = assign_ref[0, 0, :]
    n2p = s2_ref.shape[0]
    m = (a2[:, None] == lax.broadcasted_iota(jnp.int32, (blk, n2p), 1))
    contrib = lax.dot_general(m.astype(jnp.float32), g1,
                              (((0,), (0,)), ((), ())),
                              preferred_element_type=jnp.float32,
                              precision=_HI)

    @pl.when(i == 0)
    def _():
        s2_ref[...] = jnp.zeros_like(s2_ref)
    s2_ref[...] += contrib


def _gcn1_post_sep2(acc1, u1, dinv1p, b1, sep2_pad, n2p):
    blk = 512
    n1p = u1.shape[0]
    return pl.pallas_call(
        _gcn1_post_body,
        grid=(n1p // blk,),
        in_specs=[
            pl.BlockSpec((blk, D_H), lambda i: (i, 0)),
            pl.BlockSpec((blk, D_H), lambda i: (i, 0)),
            pl.BlockSpec((n1p, 1), lambda i: (0, 0)),
            pl.BlockSpec((1, D_H), lambda i: (0, 0)),
            pl.BlockSpec((1, 1, blk), lambda i: (i, 0, 0)),
        ],
        out_specs=pl.BlockSpec((n2p, D_H), lambda i: (0, 0)),
        out_shape=jax.ShapeDtypeStruct((n2p, D_H), jnp.float32),
    )(acc1, u1, dinv1p.reshape(-1, 1), b1.reshape(1, -1),
      sep2_pad.reshape(n1p // blk, 1, blk))


def _sep1_mid_body(a_ref, dinv_ref, w1_ref, batch_ref,
                   u1_ref, p0_ref):
    i = pl.program_id(0)
    blk = u1_ref.shape[0]
    xs0 = jnp.maximum(a_ref[...], 0.0)
    dv = dinv_ref[pl.ds(i * blk, blk), :]
    u1_ref[...] = dv * jnp.dot(xs0, w1_ref[...],
                               preferred_element_type=jnp.float32,
                               precision=_HI)
    b = batch_ref[0, 0, :]
    m = (b[:, None] == lax.broadcasted_iota(jnp.int32, (blk, B), 1))
    contrib = lax.dot_general(m.astype(jnp.float32), xs0,
                              (((0,), (0,)), ((), ())),
                              preferred_element_type=jnp.float32,
                              precision=_HI)

    @pl.when(i == 0)
    def _():
        p0_ref[...] = jnp.zeros_like(p0_ref)
    p0_ref[...] += contrib


def _sep1_mid(s1acc, dinv1p, w1, batch_l1_pad):
    blk = 512
    n1p = s1acc.shape[0]
    return pl.pallas_call(
        _sep1_mid_body,
        grid=(n1p // blk,),
        in_specs=[
            pl.BlockSpec((blk, D_H), lambda i: (i, 0)),
            pl.BlockSpec((n1p, 1), lambda i: (0, 0)),
            pl.BlockSpec((D_H, D_H), lambda i: (0, 0)),
            pl.BlockSpec((1, 1, blk), lambda i: (i, 0, 0)),
        ],
        out_specs=[
            pl.BlockSpec((blk, D_H), lambda i: (i, 0)),
            pl.BlockSpec((B, D_H), lambda i: (0, 0)),
        ],
        out_shape=[
            jax.ShapeDtypeStruct((n1p, D_H), jnp.float32),
            jax.ShapeDtypeStruct((B, D_H), jnp.float32),
        ],
    )(s1acc, dinv1p.reshape(-1, 1), w1,
      batch_l1_pad.reshape(n1p // blk, 1, blk))


def _final_body(a_ref, batch_ref, p0_ref,
                w1a_ref, w1b_ref, b1_ref, w2_ref, b2_ref, out_ref):
    xs1 = jnp.maximum(a_ref[...], 0.0)
    b = batch_ref[0, 0, :]
    m = (b[:, None] == lax.broadcasted_iota(jnp.int32, (xs1.shape[0], B), 1))
    p1 = lax.dot_general(m.astype(jnp.float32), xs1,
                         (((0,), (0,)), ((), ())),
                         preferred_element_type=jnp.float32, precision=_HI)
    z = jnp.dot(p0_ref[...], w1a_ref[...],
                preferred_element_type=jnp.float32, precision=_HI)
    z = z + jnp.dot(p1, w1b_ref[...],
                    preferred_element_type=jnp.float32, precision=_HI)
    z = jnp.maximum(z + b1_ref[...], 0.0)
    out_ref[...] = jnp.dot(z, w2_ref[...],
                           preferred_element_type=jnp.float32,
                           precision=_HI) + b2_ref[...]


def _final(s2acc, batch_l2_pad, p0, w1, b1, w2, b2):
    H2 = s2acc.shape[0]
    return pl.pallas_call(
        _final_body,
        grid=(1,),
        in_specs=[
            pl.BlockSpec((H2, D_H), lambda i: (0, 0)),
            pl.BlockSpec((1, 1, H2), lambda i: (0, 0, 0)),
            pl.BlockSpec((B, D_H), lambda i: (0, 0)),
            pl.BlockSpec((D_H, D_H), lambda i: (0, 0)),
            pl.BlockSpec((D_H, D_H), lambda i: (0, 0)),
            pl.BlockSpec((1, D_H), lambda i: (0, 0)),
            pl.BlockSpec((D_H, D_H), lambda i: (0, 0)),
            pl.BlockSpec((1, D_H), lambda i: (0, 0)),
        ],
        out_specs=pl.BlockSpec((B, D_H), lambda i: (0, 0)),
        out_shape=jax.ShapeDtypeStruct((B, D_H), jnp.float32),
    )(s2acc, batch_l2_pad.reshape(1, 1, -1), p0,
      w1[:D_H], w1[D_H:], b1.reshape(1, -1), w2, b2.reshape(1, -1))


E0P, E1P = 163840, 40960
N0P = 10240
N1P = 2560
N2P = 640

_deg0_k = _make_scatter_kernel(N0P, E0P, with_gather=False)
_deg1_k = _make_scatter_kernel(N1P, E1P, with_gather=False)
_gcn0_k = _make_scatter_kernel(N0P, E0P)
_gcn1_k = _make_scatter_kernel(N1P, E1P)


def _sort_edges(src, dst, e_pad, n_dst, tr):
    ne = dst.shape[0]
    order = jnp.argsort(dst)
    pad = jnp.arange(e_pad - ne, dtype=jnp.int32) % 8
    ss = jnp.concatenate([dst[order].astype(jnp.int32), n_dst + pad])
    gs = jnp.concatenate([src[order].astype(jnp.int32), pad])
    bounds = jnp.arange(33, dtype=jnp.int32) * tr
    ptr = jnp.searchsorted(ss, bounds).astype(jnp.int32)
    ptr = jnp.pad(ptr, (0, 15), constant_values=e_pad)
    return (gs.reshape(-1, 1, K_FIRE), ss.reshape(-1, 1, K_FIRE), ptr)


def kernel(x, edge_index_l0, sep1_assign, edge_index_l1, sep2_assign,
           batch_l1, batch_l2,
           enc_W1, enc_b1, enc_W2, enc_b2, enc_alpha,
           gcn_W0, gcn_b0, gcn_W1, gcn_b1,
           cls_W1, cls_b1, cls_W2, cls_b2):
    src0, dst0 = edge_index_l0[0], edge_index_l0[1]
    src1, dst1 = edge_index_l1[0], edge_index_l1[1]

    g0, s0, ptr0 = _sort_edges(src0, dst0, E0P, N0, N0P // 32)
    g1, s1, ptr1 = _sort_edges(src1, dst1, E1P, N1, N1P // 32)

    deg0 = _deg0_k(s0, ptr0)
    deg1 = _deg1_k(s1, ptr1)
    dinv0 = (deg0[:N0, 0] + 1.0) ** -0.5
    dinv1p = jnp.pad((deg1[:N1, 0] + 1.0) ** -0.5,
                     (0, N1P - N1), constant_values=1.0)
    bl1 = jnp.concatenate(
        [batch_l1.astype(jnp.int32), jnp.full((N1P - N1,), B, jnp.int32)])

    u0 = _encoder(x, enc_W1, enc_b1, enc_W2, enc_b2, enc_alpha, gcn_W0,
                  dinv0)
    acc0 = _gcn0_k(g0, s0, ptr0, u0)
    s1acc = _gcn0_post_sep1(acc0, u0, dinv0, gcn_b0, sep1_assign, N1P)
    u1, p0 = _sep1_mid(s1acc, dinv1p, gcn_W1, bl1)

    acc1 = _gcn1_k(g1, s1, ptr1, u1)
    sep2_pad = jnp.concatenate(
        [sep2_assign.astype(jnp.int32),
         jnp.full((N1P - N1,), N2P, jnp.int32)])
    s2acc = _gcn1_post_sep2(acc1, u1, dinv1p, gcn_b1, sep2_pad, N2P)

    bl2 = jnp.concatenate(
        [batch_l2.astype(jnp.int32),
         jnp.full((N2P - N2,), B, jnp.int32)])
    return _final(s2acc, bl2, p0, cls_W1, cls_b1, cls_W2, cls_b2)


# speedup vs baseline: 4.2662x; 1.0238x over previous
import functools
import jax
import jax.numpy as jnp
from jax import lax
from jax.experimental import pallas as pl
from jax.experimental.pallas import tpu as pltpu
from jax.experimental.pallas import tpu_sc as plsc

N0, N1, N2, B = 10000, 2500, 625, 16
E0, E1 = 160000, 40000
D_IN, D_MID, D_H = 768, 512, 256

NC, NS, L = 2, 16, 16

_HI = jax.lax.Precision.HIGHEST


def _round16(n):
    return (n + 15) // 16 * 16


def _chunks_of(total, step):
    out = []
    while total > 0:
        out.append(min(step, total))
        total -= out[-1]
    return out


def _pad_idx(idx, pad_to, pad_base):
    n = idx.shape[0]
    pad = pad_base + (jnp.arange(pad_to - n, dtype=jnp.int32) % 8)
    return jnp.concatenate([idx.astype(jnp.int32), pad])


def _sc_mesh():
    return plsc.VectorSubcoreMesh(core_axis_name="c", subcore_axis_name="s",
                                  num_cores=NC, num_subcores=NS)


K_FIRE = 128


def _make_scatter_kernel(n_dstp, e_pad, with_gather=True):
    K = K_FIRE
    D = D_H if with_gather else L
    TR = n_dstp // (NC * NS)
    assert TR % 8 == 0 and e_pad % K == 0

    scratch = [
        pltpu.VMEM((1, K), jnp.int32),
        pltpu.VMEM((48,), jnp.int32),
        pltpu.VMEM((TR + 8, D), jnp.float32),
        pltpu.SemaphoreType.DMA,
    ]
    if with_gather:
        scratch = [pltpu.VMEM((1, K), jnp.int32),
                   pltpu.VMEM((K, D), jnp.float32),
                   ] + scratch

    def body(*refs):
        if with_gather:
            (g_hbm, s_hbm, ptr_hbm, v_hbm, out_hbm,
             g_stage, rows, s_chunk, ptr_buf, acc, sem) = refs
        else:
            (s_hbm, ptr_hbm, out_hbm,
             s_chunk, ptr_buf, acc, sem) = refs
        c = lax.axis_index("c")
        s = lax.axis_index("s")
        w = c * NS + s
        base = w * TR

        def zrow(i, _):
            for dd in range(D // L):
                acc[i, pl.ds(dd * L, L)] = jnp.zeros((L,), jnp.float32)
            return 0
        lax.fori_loop(0, TR, zrow, 0)

        pltpu.sync_copy(ptr_hbm, ptr_buf)
        p0 = ptr_buf[pl.ds(w, L)][0]
        p1 = ptr_buf[pl.ds(w + 1, L)][0]
        j0 = p0 >> 7
        j1 = (p1 + K - 1) >> 7
        ones_vec = jnp.ones((L,), jnp.float32)

        def chunk(j, _):
            pltpu.sync_copy(s_hbm.at[j], s_chunk)
            if with_gather:
                pltpu.sync_copy(g_hbm.at[j], g_stage)
                pltpu.async_copy(v_hbm.at[g_stage.at[0]], rows, sem).wait()

            for t in range(K // L):
                dvec = s_chunk[0, pl.ds(t * L, L)]
                for lane in range(L):
                    k = t * L + lane
                    e = j * K + k
                    valid = (e >= p0) & (e < p1)
                    rr = jnp.where(valid, dvec[lane] - base,
                                   TR + (lane & 7))
                    if with_gather:
                        for dt in range(D // L):
                            plsc.addupdate(acc.at[rr, pl.ds(dt * L, L)],
                                           rows[k, pl.ds(dt * L, L)])
                    else:
                        plsc.addupdate(acc.at[rr, pl.ds(0, L)], ones_vec)
            return 0
        lax.fori_loop(j0, j1, chunk, 0)

        pltpu.sync_copy(acc.at[pl.ds(0, TR)],
                        out_hbm.at[pl.ds(base, TR)])

    return functools.partial(
        pl.kernel,
        out_type=jax.ShapeDtypeStruct((n_dstp, D), jnp.float32),
        mesh=_sc_mesh(),
        scratch_types=scratch,
    )(body)


def _encoder_body(x_ref, w1_ref, b1_ref, w2_ref, b2_ref, alpha_ref, w0_ref,
                  dinv_ref, out_ref):
    h = jnp.dot(x_ref[...], w1_ref[...], preferred_element_type=jnp.float32,
                precision=_HI)
    h = h + b1_ref[...]
    h = jnp.where(h >= 0, h, 0.01 * h)
    h = jnp.dot(h, w2_ref[...], preferred_element_type=jnp.float32,
                precision=_HI)
    h = h + b2_ref[...]
    h = jnp.where(h >= 0, h, alpha_ref[...] * h)
    p = jnp.dot(h, w0_ref[...], preferred_element_type=jnp.float32,
                precision=_HI)
    out_ref[...] = p * dinv_ref[...]


def _encoder(x, w1, b1, w2, b2, alpha, w0, dinv):
    blk = 1000
    return pl.pallas_call(
        _encoder_body,
        grid=(N0 // blk,),
        in_specs=[
            pl.BlockSpec((blk, D_IN), lambda i: (i, 0)),
            pl.BlockSpec((D_IN, D_MID), lambda i: (0, 0)),
            pl.BlockSpec((1, D_MID), lambda i: (0, 0)),
            pl.BlockSpec((D_MID, D_H), lambda i: (0, 0)),
            pl.BlockSpec((1, D_H), lambda i: (0, 0)),
            pl.BlockSpec((1, D_H), lambda i: (0, 0)),
            pl.BlockSpec((D_H, D_H), lambda i: (0, 0)),
            pl.BlockSpec((blk, 1), lambda i: (i, 0)),
        ],
        out_specs=pl.BlockSpec((blk, D_H), lambda i: (i, 0)),
        out_shape=jax.ShapeDtypeStruct((N0, D_H), jnp.float32),
    )(x, w1, b1.reshape(1, -1), w2, b2.reshape(1, -1), alpha.reshape(1, -1),
      w0, dinv.reshape(-1, 1))


def _gcn0_post_body(acc_ref, u_ref, dinv_ref, b_ref, assign_ref, s1_ref):
    i = pl.program_id(0)
    blk = acc_ref.shape[0]
    n1p = s1_ref.shape[0]
    dv = dinv_ref[pl.ds(i * blk, blk), :]
    g0 = jnp.maximum(dv * (acc_ref[...] + u_ref[...]) + b_ref[...], 0.0)
    a1 = assign_ref[0, 0, :]
    m = (a1[:, None] == lax.broadcasted_iota(jnp.int32, (blk, n1p), 1))
    contrib = lax.dot_general(m.astype(jnp.float32), g0,
                              (((0,), (0,)), ((), ())),
                              preferred_element_type=jnp.float32,
                              precision=_HI)

    @pl.when(i == 0)
    def _():
        s1_ref[...] = jnp.zeros_like(s1_ref)
    s1_ref[...] += contrib


def _gcn0_post_sep1(acc_flat, u0, dinv0, b0, sep1_assign, n1p):
    blk = 1000
    return pl.pallas_call(
        _gcn0_post_body,
        grid=(N0 // blk,),
        in_specs=[
            pl.BlockSpec((blk, D_H), lambda i: (i, 0)),
            pl.BlockSpec((blk, D_H), lambda i: (i, 0)),
            pl.BlockSpec((N0, 1), lambda i: (0, 0)),
            pl.BlockSpec((1, D_H), lambda i: (0, 0)),
            pl.BlockSpec((1, 1, blk), lambda i: (i, 0, 0)),
        ],
        out_specs=pl.BlockSpec((n1p, D_H), lambda i: (0, 0)),
        out_shape=jax.ShapeDtypeStruct((n1p, D_H), jnp.float32),
    )(acc_flat, u0, dinv0.reshape(-1, 1), b0.reshape(1, -1),
      sep1_assign.astype(jnp.int32).reshape(N0 // blk, 1, blk))


def _gcn1_post_body(a_ref, u_ref, dinv_ref, b_ref, assign_ref, s2_ref):
    i = pl.program_id(0)
    blk = a_ref.shape[0]
    dv = dinv_ref[pl.ds(i * blk, blk), :]
    g1 = jnp.maximum(dv * (a_ref[...] + u_ref[...]) + b_ref[...], 0.0)
    a2 = assign_ref[0, 0, :]
    n2p = s2_ref.shape[0]
    m = (a2[:, None] == lax.broadcasted_iota(jnp.int32, (blk, n2p), 1))
    contrib = lax.dot_general(m.astype(jnp.float32), g1,
                              (((0,), (0,)), ((), ())),
                              preferred_element_type=jnp.float32,
                              precision=_HI)

    @pl.when(i == 0)
    def _():
        s2_ref[...] = jnp.zeros_like(s2_ref)
    s2_ref[...] += contrib


def _gcn1_post_sep2(acc1, u1, dinv1p, b1, sep2_pad, n2p):
    blk = 512
    n1p = u1.shape[0]
    return pl.pallas_call(
        _gcn1_post_body,
        grid=(n1p // blk,),
        in_specs=[
            pl.BlockSpec((blk, D_H), lambda i: (i, 0)),
            pl.BlockSpec((blk, D_H), lambda i: (i, 0)),
            pl.BlockSpec((n1p, 1), lambda i: (0, 0)),
            pl.BlockSpec((1, D_H), lambda i: (0, 0)),
            pl.BlockSpec((1, 1, blk), lambda i: (i, 0, 0)),
        ],
        out_specs=pl.BlockSpec((n2p, D_H), lambda i: (0, 0)),
        out_shape=jax.ShapeDtypeStruct((n2p, D_H), jnp.float32),
    )(acc1, u1, dinv1p.reshape(-1, 1), b1.reshape(1, -1),
      sep2_pad.reshape(n1p // blk, 1, blk))


def _sep1_mid_body(a_ref, dinv_ref, w1_ref, batch_ref,
                   u1_ref, p0_ref):
    i = pl.program_id(0)
    blk = u1_ref.shape[0]
    xs0 = jnp.maximum(a_ref[...], 0.0)
    dv = dinv_ref[pl.ds(i * blk, blk), :]
    u1_ref[...] = dv * jnp.dot(xs0, w1_ref[...],
                               preferred_element_type=jnp.float32,
                               precision=_HI)
    b = batch_ref[0, 0, :]
    m = (b[:, None] == lax.broadcasted_iota(jnp.int32, (blk, B), 1))
    contrib = lax.dot_general(m.astype(jnp.float32), xs0,
                              (((0,), (0,)), ((), ())),
                              preferred_element_type=jnp.float32,
                              precision=_HI)

    @pl.when(i == 0)
    def _():
        p0_ref[...] = jnp.zeros_like(p0_ref)
    p0_ref[...] += contrib


def _sep1_mid(s1acc, dinv1p, w1, batch_l1_pad):
    blk = 512
    n1p = s1acc.shape[0]
    return pl.pallas_call(
        _sep1_mid_body,
        grid=(n1p // blk,),
        in_specs=[
            pl.BlockSpec((blk, D_H), lambda i: (i, 0)),
            pl.BlockSpec((n1p, 1), lambda i: (0, 0)),
            pl.BlockSpec((D_H, D_H), lambda i: (0, 0)),
            pl.BlockSpec((1, 1, blk), lambda i: (i, 0, 0)),
        ],
        out_specs=[
            pl.BlockSpec((blk, D_H), lambda i: (i, 0)),
            pl.BlockSpec((B, D_H), lambda i: (0, 0)),
        ],
        out_shape=[
            jax.ShapeDtypeStruct((n1p, D_H), jnp.float32),
            jax.ShapeDtypeStruct((B, D_H), jnp.float32),
        ],
    )(s1acc, dinv1p.reshape(-1, 1), w1,
      batch_l1_pad.reshape(n1p // blk, 1, blk))


def _final_body(a_ref, batch_ref, p0_ref,
                w1a_ref, w1b_ref, b1_ref, w2_ref, b2_ref, out_ref):
    xs1 = jnp.maximum(a_ref[...], 0.0)
    b = batch_ref[0, 0, :]
    m = (b[:, None] == lax.broadcasted_iota(jnp.int32, (xs1.shape[0], B), 1))
    p1 = lax.dot_general(m.astype(jnp.float32), xs1,
                         (((0,), (0,)), ((), ())),
                         preferred_element_type=jnp.float32, precision=_HI)
    z = jnp.dot(p0_ref[...], w1a_ref[...],
                preferred_element_type=jnp.float32, precision=_HI)
    z = z + jnp.dot(p1, w1b_ref[...],
                    preferred_element_type=jnp.float32, precision=_HI)
    z = jnp.maximum(z + b1_ref[...], 0.0)
    out_ref[...] = jnp.dot(z, w2_ref[...],
                           preferred_element_type=jnp.float32,
                           precision=_HI) + b2_ref[...]


def _final(s2acc, batch_l2_pad, p0, w1, b1, w2, b2):
    H2 = s2acc.shape[0]
    return pl.pallas_call(
        _final_body,
        grid=(1,),
        in_specs=[
            pl.BlockSpec((H2, D_H), lambda i: (0, 0)),
            pl.BlockSpec((1, 1, H2), lambda i: (0, 0, 0)),
            pl.BlockSpec((B, D_H), lambda i: (0, 0)),
            pl.BlockSpec((D_H, D_H), lambda i: (0, 0)),
            pl.BlockSpec((D_H, D_H), lambda i: (0, 0)),
            pl.BlockSpec((1, D_H), lambda i: (0, 0)),
            pl.BlockSpec((D_H, D_H), lambda i: (0, 0)),
            pl.BlockSpec((1, D_H), lambda i: (0, 0)),
        ],
        out_specs=pl.BlockSpec((B, D_H), lambda i: (0, 0)),
        out_shape=jax.ShapeDtypeStruct((B, D_H), jnp.float32),
    )(s2acc, batch_l2_pad.reshape(1, 1, -1), p0,
      w1[:D_H], w1[D_H:], b1.reshape(1, -1), w2, b2.reshape(1, -1))


E0P, E1P = 163840, 40960
N0P = 10240
N1P = 2560
N2P = 640

_deg0_k = _make_scatter_kernel(N0P, E0P, with_gather=False)
_deg1_k = _make_scatter_kernel(N1P, E1P, with_gather=False)
_gcn0_k = _make_scatter_kernel(N0P, E0P)
_gcn1_k = _make_scatter_kernel(N1P, E1P)


def _sort_edges(src, dst, e_pad, n_dst, tr):
    ne = dst.shape[0]
    order = jnp.argsort(dst)
    pad = jnp.arange(e_pad - ne, dtype=jnp.int32) % 8
    ss = jnp.concatenate([dst[order].astype(jnp.int32), n_dst + pad])
    gs = jnp.concatenate([src[order].astype(jnp.int32), pad])
    bounds = jnp.arange(33, dtype=jnp.int32) * tr
    ptr = jnp.searchsorted(ss, bounds).astype(jnp.int32)
    ptr = jnp.pad(ptr, (0, 15), constant_values=e_pad)
    return (gs.reshape(-1, 1, K_FIRE), ss.reshape(-1, 1, K_FIRE), ptr)


def kernel(x, edge_index_l0, sep1_assign, edge_index_l1, sep2_assign,
           batch_l1, batch_l2,
           enc_W1, enc_b1, enc_W2, enc_b2, enc_alpha,
           gcn_W0, gcn_b0, gcn_W1, gcn_b1,
           cls_W1, cls_b1, cls_W2, cls_b2):
    src0, dst0 = edge_index_l0[0], edge_index_l0[1]
    src1, dst1 = edge_index_l1[0], edge_index_l1[1]

    g0, s0, ptr0 = _sort_edges(src0, dst0, E0P, N0, N0P // 32)
    g1, s1, ptr1 = _sort_edges(src1, dst1, E1P, N1, N1P // 32)

    deg0 = _deg0_k(s0, ptr0)
    deg1 = _deg1_k(s1, ptr1)
    dinv0 = (deg0[:N0, 0] + 1.0) ** -0.5
    dinv1p = jnp.pad((deg1[:N1, 0] + 1.0) ** -0.5,
                     (0, N1P - N1), constant_values=1.0)
    bl1 = jnp.concatenate(
        [batch_l1.astype(jnp.int32), jnp.full((N1P - N1,), B, jnp.int32)])

    u0 = _encoder(x, enc_W1, enc_b1, enc_W2, enc_b2, enc_alpha, gcn_W0,
                  dinv0)
    acc0 = _gcn0_k(g0, s0, ptr0, u0)
    s1acc = _gcn0_post_sep1(acc0, u0, dinv0, gcn_b0, sep1_assign, N1P)
    u1, p0 = _sep1_mid(s1acc, dinv1p, gcn_W1, bl1)

    acc1 = _gcn1_k(g1, s1, ptr1, u1)
    sep2_pad = jnp.concatenate(
        [sep2_assign.astype(jnp.int32),
         jnp.full((N1P - N1,), N2P, jnp.int32)])
    s2acc = _gcn1_post_sep2(acc1, u1, dinv1p, gcn_b1, sep2_pad, N2P)

    bl2 = jnp.concatenate(
        [batch_l2.astype(jnp.int32),
         jnp.full((N2P - N2,), B, jnp.int32)])
    return _final(s2acc, bl2, p0, cls_W1, cls_b1, cls_W2, cls_b2)
